# Initial kernel scaffold; baseline (speedup 1.0000x reference)
import jax
import jax.numpy as jnp
from jax.experimental import pallas as pl


def kernel(A0, A1, feats0, feats1, out0, out1, src0, dst0, src1, dst1, p1w1_0, p1b1_0, p1w2_0, p1b2_0, p2w1_0, p2b1_0, p2w2_0, p2b2_0, p1w1_1, p1b1_1, p1w2_1, p1b2_1, p2w1_1, p2b1_1, p2w2_1, p2b2_1):
    raise NotImplementedError("write your pallas kernel here")


# speedup vs baseline: 1.1804x; 1.1804x over previous
import functools

import jax
import jax.numpy as jnp
from jax import lax
from jax.experimental import pallas as pl
from jax.experimental.pallas import tpu as pltpu
from jax.experimental.pallas import tpu_sc as plsc

_BETA = 0.5
_N = 4096
_B = 2048
_NE = 2 * _B
_NC = 2
_NS = 16
_NW = _NC * _NS
_EPW = _NE // _NW
_GB = 2
_NB = _EPW // _GB
_HALF = _N // 2
_LANES = 16
_CH = _HALF // _LANES
_UNROLL = 4


def _sc_partials(A0v, A1v, idx, w0, w1):
    mesh = plsc.VectorSubcoreMesh(core_axis_name="c", subcore_axis_name="s")

    @functools.partial(
        pl.kernel,
        out_type=jax.ShapeDtypeStruct((_NE, 6 * _LANES), jnp.float32),
        mesh=mesh,
        scratch_types=[
            pltpu.VMEM((_GB * 4, _HALF), jnp.float32),
            pltpu.VMEM((_GB * 4, _HALF), jnp.float32),
            pltpu.VMEM((_GB * 4, _HALF), jnp.float32),
            pltpu.VMEM((_GB * 4, _HALF), jnp.float32),
            pltpu.VMEM((_N,), jnp.float32),
            pltpu.VMEM((_N,), jnp.float32),
            pltpu.VMEM((_NB, 4 * _GB), jnp.int32),
            pltpu.VMEM((_EPW, 6 * _LANES), jnp.float32),
            pltpu.SemaphoreType.DMA,
            pltpu.SemaphoreType.DMA,
        ],
    )
    def body(A0h, A1h, idxh, w0h, w1h, out_h,
             a0b0, a0b1, a1b0, a1b1, wv0, wv1, idx_v, res_v, sem0, sem1):
        wid = lax.axis_index("s") * _NC + lax.axis_index("c")
        bufs = ((a0b0, a1b0, sem0), (a0b1, a1b1, sem1))

        pltpu.sync_copy(idxh.at[wid], idx_v)
        pltpu.sync_copy(w0h, wv0)
        pltpu.sync_copy(w1h, wv1)

        def gathers(p, b):
            b0, b1, sem = bufs[p]
            return (pltpu.make_async_copy(A0h.at[idx_v.at[b]], b0, sem),
                    pltpu.make_async_copy(A1h.at[idx_v.at[b]], b1, sem))

        def start(p, b):
            g0, g1 = gathers(p, b)
            g0.start()
            g1.start()

        def wait(p, b):
            g0, g1 = gathers(p, b)
            g0.wait()
            g1.wait()

        start(0, 0)
        start(1, 1)

        zeros = jnp.zeros((_LANES,), jnp.float32)

        def edge_accs(b0, b1, i):
            accs = (zeros,) * 6
            for h in (0, 1):
                def chunk(k, accs, h=h):
                    f, r, s0s, s1s, s0d, s1d = accs
                    for u in range(_UNROLL):
                        kk = k * _UNROLL + u
                        csl = pl.ds(kk * _LANES, _LANES)
                        wsl = pl.ds(h * _HALF + kk * _LANES, _LANES)
                        w0c = wv0[wsl]
                        w1c = wv1[wsl]
                        b0s = b0[4 * i + h, csl] * w0c
                        b1s = b1[4 * i + h, csl] * w1c
                        b0d = b0[4 * i + 2 + h, csl] * w0c
                        b1d = b1[4 * i + 2 + h, csl] * w1c
                        f = f + b0s * b1d
                        r = r + b1s * b0d
                        s0s = s0s + b0s * b0s
                        s1s = s1s + b1s * b1s
                        s0d = s0d + b0d * b0d
                        s1d = s1d + b1d * b1d
                    return (f, r, s0s, s1s, s0d, s1d)
                accs = lax.fori_loop(0, _CH // _UNROLL, chunk, accs)
            return accs

        def run_batch(p, b):
            wait(p, b)
            b0, b1, _ = bufs[p][:3]
            for i in range(_GB):
                accs = edge_accs(b0, b1, i)
                e_loc = b * _GB + i
                for q in range(6):
                    res_v[e_loc, pl.ds(q * _LANES, _LANES)] = accs[q]
            nb = b + 2

            @pl.when(nb < _NB)
            def _():
                start(p, nb)

        def g_body(g, carry):
            run_batch(0, 2 * g)
            run_batch(1, 2 * g + 1)
            return carry

        lax.fori_loop(0, _NB // 2, g_body, 0)
        pltpu.sync_copy(res_v, out_h.at[pl.ds(wid * _EPW, _EPW)])

    return body(A0v, A1v, idx, w0, w1)


def _finisher(P, mw):
    def body(p_ref, w10_ref, b10_ref, w20_ref, b20_ref,
             v10_ref, c10_ref, v20_ref, c20_ref,
             w11_ref, b11_ref, w21_ref, b21_ref,
             v11_ref, c11_ref, v21_ref, c21_ref, o0_ref, o1_ref):
        p = p_ref[...]
        s = [jnp.sum(p[:, 16 * q:16 * q + 16], axis=1, keepdims=True)
             for q in range(6)]
        f, r, s0s, s1s, s0d, s1d = s
        df = jnp.sqrt(s0s) * jnp.sqrt(s1d) + 1e-30
        dr = jnp.sqrt(s1s) * jnp.sqrt(s0d) + 1e-30
        on = f / df + r / dr

        def bf(x):
            return x.astype(jnp.bfloat16).astype(jnp.float32)

        def mlp(x, w1, b1, w2, b2):
            h = jnp.maximum(x * w1[...] + b1[...], 0.0)
            return (jnp.sum(bf(h) * bf(w2[...]), axis=1, keepdims=True)
                    + b2[...])

        on0 = on[:_B]
        on1 = on[_B:]
        sl0 = _BETA * mlp(on0, w10_ref, b10_ref, w20_ref, b20_ref)
        o0_ref[...] = mlp(sl0, v10_ref, c10_ref, v20_ref, c20_ref)
        sl1 = _BETA * mlp(on1, w11_ref, b11_ref, w21_ref, b21_ref)
        o1_ref[...] = mlp(sl1, v11_ref, c11_ref, v21_ref, c21_ref)

    out = pl.pallas_call(
        body,
        out_shape=(jax.ShapeDtypeStruct((_B, 1), jnp.float32),
                   jax.ShapeDtypeStruct((_B, 1), jnp.float32)),
    )(P, *mw)
    return out


def kernel(A0, A1, feats0, feats1, out0, out1, src0, dst0, src1, dst1,
           p1w1_0, p1b1_0, p1w2_0, p1b2_0, p2w1_0, p2b1_0, p2w2_0, p2b2_0,
           p1w1_1, p1b1_1, p1w2_1, p1b2_1, p2w1_1, p2b1_1, p2w2_1, p2b2_1):
    A0v = A0.reshape(2 * _N, _HALF)
    A1v = A1.reshape(2 * _N, _HALF)
    src = jnp.concatenate([src0, src1])
    dst = jnp.concatenate([dst0, dst1])
    idx = jnp.stack([2 * src, 2 * src + 1, 2 * dst, 2 * dst + 1], axis=1)
    idx = idx.reshape(_NW, _NB, 4 * _GB).astype(jnp.int32)
    w0 = feats0.reshape(_N)
    w1 = feats1.reshape(_N)

    P = _sc_partials(A0v, A1v, idx, w0, w1)

    mw = (p1w1_0, p1b1_0.reshape(1, 64), p1w2_0.reshape(1, 64),
          p1b2_0.reshape(1, 1),
          p2w1_0, p2b1_0.reshape(1, 64), p2w2_0.reshape(1, 64),
          p2b2_0.reshape(1, 1),
          p1w1_1, p1b1_1.reshape(1, 64), p1w2_1.reshape(1, 64),
          p1b2_1.reshape(1, 1),
          p2w1_1, p2b1_1.reshape(1, 64), p2w2_1.reshape(1, 64),
          p2b2_1.reshape(1, 1))
    o0, o1 = _finisher(P, mw)
    return jnp.stack([o0, o1], axis=0)


# speedup vs baseline: 2.0782x; 1.7606x over previous
import functools

import jax
import jax.numpy as jnp
from jax import lax
from jax.experimental import pallas as pl
from jax.experimental.pallas import tpu as pltpu
from jax.experimental.pallas import tpu_sc as plsc

_BETA = 0.5
_N = 4096
_B = 2048
_NE = 2 * _B
_NC = 2
_NS = 16
_NW = _NC * _NS
_EPW = _NE // _NW
_GB = 2
_NB = _EPW // _GB
_LANES = 16
_CH = _N // _LANES
_UNROLL = 4


def _sc_partials(A0v, A1v, idx, w0, w1):
    mesh = plsc.VectorSubcoreMesh(core_axis_name="c", subcore_axis_name="s")

    @functools.partial(
        pl.kernel,
        out_type=jax.ShapeDtypeStruct((_NE, 6 * _LANES), jnp.float32),
        mesh=mesh,
        scratch_types=[
            pltpu.VMEM((_GB * 2, _N), jnp.float32),
            pltpu.VMEM((_GB * 2, _N), jnp.float32),
            pltpu.VMEM((_GB * 2, _N), jnp.float32),
            pltpu.VMEM((_GB * 2, _N), jnp.float32),
            pltpu.VMEM((_N,), jnp.float32),
            pltpu.VMEM((_N,), jnp.float32),
            pltpu.VMEM((_NB, 2 * _GB), jnp.int32),
            pltpu.VMEM((_EPW, 6 * _LANES), jnp.float32),
            pltpu.SemaphoreType.DMA,
            pltpu.SemaphoreType.DMA,
        ],
    )
    def body(A0h, A1h, idxh, w0h, w1h, out_h,
             a0b0, a0b1, a1b0, a1b1, wv0, wv1, idx_v, res_v, sem0, sem1):
        wid = lax.axis_index("s") * _NC + lax.axis_index("c")
        bufs = ((a0b0, a1b0, sem0), (a0b1, a1b1, sem1))

        pltpu.sync_copy(idxh.at[wid], idx_v)
        pltpu.sync_copy(w0h, wv0)
        pltpu.sync_copy(w1h, wv1)

        def gathers(p, b):
            b0, b1, sem = bufs[p]
            return (pltpu.make_async_copy(A0h.at[idx_v.at[b]], b0, sem),
                    pltpu.make_async_copy(A1h.at[idx_v.at[b]], b1, sem))

        def start(p, b):
            g0, g1 = gathers(p, b)
            g0.start()
            g1.start()

        def wait(p, b):
            g0, g1 = gathers(p, b)
            g0.wait()
            g1.wait()

        start(0, 0)
        start(1, 1)

        zeros = jnp.zeros((_LANES,), jnp.float32)

        def edge_accs(b0, b1, i):
            def chunk(k, accs):
                f, r, s0s, s1s, s0d, s1d = accs
                for u in range(_UNROLL):
                    csl = pl.ds((k * _UNROLL + u) * _LANES, _LANES)
                    w0c = wv0[csl]
                    w1c = wv1[csl]
                    b0s = b0[2 * i, csl] * w0c
                    b1s = b1[2 * i, csl] * w1c
                    b0d = b0[2 * i + 1, csl] * w0c
                    b1d = b1[2 * i + 1, csl] * w1c
                    f = f + b0s * b1d
                    r = r + b1s * b0d
                    s0s = s0s + b0s * b0s
                    s1s = s1s + b1s * b1s
                    s0d = s0d + b0d * b0d
                    s1d = s1d + b1d * b1d
                return (f, r, s0s, s1s, s0d, s1d)
            return lax.fori_loop(0, _CH // _UNROLL, chunk, (zeros,) * 6)

        def run_batch(p, b):
            wait(p, b)
            b0, b1, _ = bufs[p][:3]
            for i in range(_GB):
                accs = edge_accs(b0, b1, i)
                e_loc = b * _GB + i
                for q in range(6):
                    res_v[e_loc, pl.ds(q * _LANES, _LANES)] = accs[q]
            nb = b + 2

            @pl.when(nb < _NB)
            def _():
                start(p, nb)

        def g_body(g, carry):
            run_batch(0, 2 * g)
            run_batch(1, 2 * g + 1)
            return carry

        lax.fori_loop(0, _NB // 2, g_body, 0)
        pltpu.sync_copy(res_v, out_h.at[pl.ds(wid * _EPW, _EPW)])

    return body(A0v, A1v, idx, w0, w1)


def _finisher(P, mw):
    def body(p_ref, w10_ref, b10_ref, w20_ref, b20_ref,
             v10_ref, c10_ref, v20_ref, c20_ref,
             w11_ref, b11_ref, w21_ref, b21_ref,
             v11_ref, c11_ref, v21_ref, c21_ref, o0_ref, o1_ref):
        p = p_ref[...]
        s = [jnp.sum(p[:, 16 * q:16 * q + 16], axis=1, keepdims=True)
             for q in range(6)]
        f, r, s0s, s1s, s0d, s1d = s
        df = jnp.sqrt(s0s) * jnp.sqrt(s1d) + 1e-30
        dr = jnp.sqrt(s1s) * jnp.sqrt(s0d) + 1e-30
        on = f / df + r / dr

        def bf(x):
            return x.astype(jnp.bfloat16).astype(jnp.float32)

        def mlp(x, w1, b1, w2, b2):
            h = jnp.maximum(x * w1[...] + b1[...], 0.0)
            return (jnp.sum(bf(h) * bf(w2[...]), axis=1, keepdims=True)
                    + b2[...])

        on0 = on[:_B]
        on1 = on[_B:]
        sl0 = _BETA * mlp(on0, w10_ref, b10_ref, w20_ref, b20_ref)
        o0_ref[...] = mlp(sl0, v10_ref, c10_ref, v20_ref, c20_ref)
        sl1 = _BETA * mlp(on1, w11_ref, b11_ref, w21_ref, b21_ref)
        o1_ref[...] = mlp(sl1, v11_ref, c11_ref, v21_ref, c21_ref)

    out = pl.pallas_call(
        body,
        out_shape=(jax.ShapeDtypeStruct((_B, 1), jnp.float32),
                   jax.ShapeDtypeStruct((_B, 1), jnp.float32)),
    )(P, *mw)
    return out


def kernel(A0, A1, feats0, feats1, out0, out1, src0, dst0, src1, dst1,
           p1w1_0, p1b1_0, p1w2_0, p1b2_0, p2w1_0, p2b1_0, p2w2_0, p2b2_0,
           p1w1_1, p1b1_1, p1w2_1, p1b2_1, p2w1_1, p2b1_1, p2w2_1, p2b2_1):
    src = jnp.concatenate([src0, src1])
    dst = jnp.concatenate([dst0, dst1])
    idx = jnp.stack([src, dst], axis=1)
    idx = idx.reshape(_NW, _NB, 2 * _GB).astype(jnp.int32)
    w0 = feats0.reshape(_N)
    w1 = feats1.reshape(_N)

    P = _sc_partials(A0, A1, idx, w0, w1)

    mw = (p1w1_0, p1b1_0.reshape(1, 64), p1w2_0.reshape(1, 64),
          p1b2_0.reshape(1, 1),
          p2w1_0, p2b1_0.reshape(1, 64), p2w2_0.reshape(1, 64),
          p2b2_0.reshape(1, 1),
          p1w1_1, p1b1_1.reshape(1, 64), p1w2_1.reshape(1, 64),
          p1b2_1.reshape(1, 1),
          p2w1_1, p2b1_1.reshape(1, 64), p2w2_1.reshape(1, 64),
          p2b2_1.reshape(1, 1))
    o0, o1 = _finisher(P, mw)
    return jnp.stack([o0, o1], axis=0)


# speedup vs baseline: 2.1296x; 1.0247x over previous
import functools

import jax
import jax.numpy as jnp
from jax import lax
from jax.experimental import pallas as pl
from jax.experimental.pallas import tpu as pltpu
from jax.experimental.pallas import tpu_sc as plsc

_BETA = 0.5
_N = 4096
_B = 2048
_NE = 2 * _B
_NC = 2
_NS = 16
_NW = _NC * _NS
_EPW = _NE // _NW
_GB = 2
_NB = _EPW // _GB
_LANES = 16
_CH = _N // _LANES
_UNROLL = 4


def _sc_partials(A0v, A1v, idx, w0, w1):
    mesh = plsc.VectorSubcoreMesh(core_axis_name="c", subcore_axis_name="s")

    @functools.partial(
        pl.kernel,
        out_type=jax.ShapeDtypeStruct((_NE, _LANES), jnp.float32),
        mesh=mesh,
        scratch_types=[
            pltpu.VMEM((_GB * 2, _N), jnp.float32),
            pltpu.VMEM((_GB * 2, _N), jnp.float32),
            pltpu.VMEM((_GB * 2, _N), jnp.float32),
            pltpu.VMEM((_GB * 2, _N), jnp.float32),
            pltpu.VMEM((_N,), jnp.float32),
            pltpu.VMEM((_N,), jnp.float32),
            pltpu.VMEM((_NB, 2 * _GB), jnp.int32),
            pltpu.VMEM((_EPW, _LANES), jnp.float32),
            pltpu.SemaphoreType.DMA,
            pltpu.SemaphoreType.DMA,
        ],
    )
    def body(A0h, A1h, idxh, w0h, w1h, out_h,
             a0b0, a0b1, a1b0, a1b1, wv0, wv1, idx_v, res_v, sem0, sem1):
        wid = lax.axis_index("s") * _NC + lax.axis_index("c")
        bufs = ((a0b0, a1b0, sem0), (a0b1, a1b1, sem1))

        pltpu.sync_copy(idxh.at[wid], idx_v)
        pltpu.sync_copy(w0h, wv0)
        pltpu.sync_copy(w1h, wv1)

        def gathers(p, b):
            b0, b1, sem = bufs[p]
            return (pltpu.make_async_copy(A0h.at[idx_v.at[b]], b0, sem),
                    pltpu.make_async_copy(A1h.at[idx_v.at[b]], b1, sem))

        def start(p, b):
            g0, g1 = gathers(p, b)
            g0.start()
            g1.start()

        def wait(p, b):
            g0, g1 = gathers(p, b)
            g0.wait()
            g1.wait()

        start(0, 0)
        start(1, 1)

        zeros = jnp.zeros((_LANES,), jnp.float32)

        def edge_accs(b0, b1, i):
            def chunk(k, accs):
                f, r, s0s, s1s, s0d, s1d = accs
                for u in range(_UNROLL):
                    csl = pl.ds((k * _UNROLL + u) * _LANES, _LANES)
                    w0c = wv0[csl]
                    w1c = wv1[csl]
                    b0s = b0[2 * i, csl] * w0c
                    b1s = b1[2 * i, csl] * w1c
                    b0d = b0[2 * i + 1, csl] * w0c
                    b1d = b1[2 * i + 1, csl] * w1c
                    f = f + b0s * b1d
                    r = r + b1s * b0d
                    s0s = s0s + b0s * b0s
                    s1s = s1s + b1s * b1s
                    s0d = s0d + b0d * b0d
                    s1d = s1d + b1d * b1d
                return (f, r, s0s, s1s, s0d, s1d)
            return lax.fori_loop(0, _CH // _UNROLL, chunk, (zeros,) * 6)

        def run_batch(p, b):
            wait(p, b)
            b0, b1, _ = bufs[p][:3]
            lanes = lax.iota(jnp.int32, _LANES)

            def allsum(x):
                for s in (8, 4, 2, 1):
                    x = x + x.at[jnp.bitwise_xor(lanes, s)].get(
                        mode="promise_in_bounds")
                return x

            for i in range(_GB):
                accs = edge_accs(b0, b1, i)
                e_loc = b * _GB + i
                vec = zeros
                for q in range(6):
                    vec = jnp.where(lanes == q, allsum(accs[q]), vec)
                res_v[e_loc, :] = vec
            nb = b + 2

            @pl.when(nb < _NB)
            def _():
                start(p, nb)

        def g_body(g, carry):
            run_batch(0, 2 * g)
            run_batch(1, 2 * g + 1)
            return carry

        lax.fori_loop(0, _NB // 2, g_body, 0)
        pltpu.sync_copy(res_v, out_h.at[pl.ds(wid * _EPW, _EPW)])

    return body(A0v, A1v, idx, w0, w1)


def _finisher(P, mw):
    def body(p_ref, w10_ref, b10_ref, w20_ref, b20_ref,
             v10_ref, c10_ref, v20_ref, c20_ref,
             w11_ref, b11_ref, w21_ref, b21_ref,
             v11_ref, c11_ref, v21_ref, c21_ref, o_ref):
        p = p_ref[...]
        f, r, s0s, s1s, s0d, s1d = [p[:, q:q + 1] for q in range(6)]
        df = jnp.sqrt(s0s) * jnp.sqrt(s1d) + 1e-30
        dr = jnp.sqrt(s1s) * jnp.sqrt(s0d) + 1e-30
        on = f / df + r / dr

        def bf(x):
            return x.astype(jnp.bfloat16).astype(jnp.float32)

        def mlp(x, w1, b1, w2, b2):
            h = jnp.maximum(x * w1[...] + b1[...], 0.0)
            return (jnp.sum(bf(h) * bf(w2[...]), axis=1, keepdims=True)
                    + b2[...])

        on0 = on[:_B]
        on1 = on[_B:]
        sl0 = _BETA * mlp(on0, w10_ref, b10_ref, w20_ref, b20_ref)
        o_ref[:_B] = mlp(sl0, v10_ref, c10_ref, v20_ref, c20_ref)
        sl1 = _BETA * mlp(on1, w11_ref, b11_ref, w21_ref, b21_ref)
        o_ref[_B:] = mlp(sl1, v11_ref, c11_ref, v21_ref, c21_ref)

    out = pl.pallas_call(
        body,
        out_shape=jax.ShapeDtypeStruct((_NE, 1), jnp.float32),
    )(P, *mw)
    return out


def kernel(A0, A1, feats0, feats1, out0, out1, src0, dst0, src1, dst1,
           p1w1_0, p1b1_0, p1w2_0, p1b2_0, p2w1_0, p2b1_0, p2w2_0, p2b2_0,
           p1w1_1, p1b1_1, p1w2_1, p1b2_1, p2w1_1, p2b1_1, p2w2_1, p2b2_1):
    src = jnp.concatenate([src0, src1])
    dst = jnp.concatenate([dst0, dst1])
    idx = jnp.stack([src, dst], axis=1)
    idx = idx.reshape(_NW, _NB, 2 * _GB).astype(jnp.int32)
    w0 = feats0.reshape(_N)
    w1 = feats1.reshape(_N)

    P = _sc_partials(A0, A1, idx, w0, w1)

    mw = (p1w1_0, p1b1_0.reshape(1, 64), p1w2_0.reshape(1, 64),
          p1b2_0.reshape(1, 1),
          p2w1_0, p2b1_0.reshape(1, 64), p2w2_0.reshape(1, 64),
          p2b2_0.reshape(1, 1),
          p1w1_1, p1b1_1.reshape(1, 64), p1w2_1.reshape(1, 64),
          p1b2_1.reshape(1, 1),
          p2w1_1, p2b1_1.reshape(1, 64), p2w2_1.reshape(1, 64),
          p2b2_1.reshape(1, 1))
    out = _finisher(P, mw)
    return out.reshape(2, _B, 1)
